# baseline (speedup 1.0000x reference)
import functools

import jax
import jax.numpy as jnp
from jax import lax
from jax.experimental import pallas as pl
from jax.experimental.pallas import tpu as pltpu
from jax.experimental.pallas import tpu_sc as plsc

_HIDDEN = 768
_E = 64
_BM = 4096
_L = 16
_NW = 32


def _logits_body(x_ref, w_ref, b_ref, out_ref):
    w = w_ref[...]
    b = b_ref[...]
    spw = out_ref.shape[0]
    tpw = out_ref.shape[2]
    for j in range(spw):
        xj = x_ref[pl.ds(j * tpw, tpw), :]
        logits = jax.lax.dot_general(
            w, xj, (((1,), (1,)), ((), ())),
            preferred_element_type=jnp.float32,
        )
        out_ref[j] = logits + b


def _tc_logits(x, weight, bias):
    n_tok = x.shape[0]
    tpw = n_tok // _NW
    spw = _BM // tpw
    b2 = bias.reshape(_E, 1)
    grid = (n_tok // _BM,)
    return pl.pallas_call(
        _logits_body,
        grid=grid,
        in_specs=[
            pl.BlockSpec((_BM, _HIDDEN), lambda i: (i, 0)),
            pl.BlockSpec((_E, _HIDDEN), lambda i: (0, 0)),
            pl.BlockSpec((_E, 1), lambda i: (0, 0)),
        ],
        out_specs=pl.BlockSpec((spw, _E, tpw), lambda i: (i, 0, 0)),
        out_shape=jax.ShapeDtypeStruct((_NW, _E, tpw), jnp.float32),
        compiler_params=pltpu.CompilerParams(
            dimension_semantics=("arbitrary",),
        ),
    )(x, weight, b2)


def _sc_topk_body(lg_hbm, w1_hbm, w2_hbm, i1_hbm, i2_hbm,
                  buf, w1b, w2b, i1b, i2b):
    nc = 2
    wid = lax.axis_index("s") * nc + lax.axis_index("c")
    tpw = buf.shape[1]
    base = wid * tpw
    pltpu.sync_copy(lg_hbm.at[wid, :, :], buf)

    gb = 4

    def group(g, _):
        offs = [g * (gb * _L) + j * _L for j in range(gb)]
        m1 = [buf[0, pl.ds(o, _L)] for o in offs]
        i1 = [jnp.zeros((_L,), jnp.int32) for _ in offs]
        m2 = [jnp.full((_L,), -jnp.inf, jnp.float32) for _ in offs]
        i2 = [jnp.zeros((_L,), jnp.int32) for _ in offs]
        for e in range(1, _E):
            ev = jnp.full((_L,), e, jnp.int32)
            for j in range(gb):
                v = buf[e, pl.ds(offs[j], _L)]
                gt1 = v > m1[j]
                gt2 = v > m2[j]
                m2[j] = jnp.where(gt1, m1[j], jnp.where(gt2, v, m2[j]))
                i2[j] = jnp.where(gt1, i1[j], jnp.where(gt2, ev, i2[j]))
                m1[j] = jnp.where(gt1, v, m1[j])
                i1[j] = jnp.where(gt1, ev, i1[j])
        for j in range(gb):
            ex = jnp.exp(m2[j] - m1[j])
            w1 = 1.0 / (1.0 + ex)
            w1b[pl.ds(offs[j], _L)] = w1
            w2b[pl.ds(offs[j], _L)] = 1.0 - w1
            i1b[pl.ds(offs[j], _L)] = i1[j]
            i2b[pl.ds(offs[j], _L)] = i2[j]
        return 0

    lax.fori_loop(0, tpw // (gb * _L), group, 0)
    pltpu.sync_copy(w1b, w1_hbm.at[pl.ds(base, tpw)])
    pltpu.sync_copy(w2b, w2_hbm.at[pl.ds(base, tpw)])
    pltpu.sync_copy(i1b, i1_hbm.at[pl.ds(base, tpw)])
    pltpu.sync_copy(i2b, i2_hbm.at[pl.ds(base, tpw)])


def _sc_topk(logits_t):
    tpw = logits_t.shape[2]
    n_tok = logits_t.shape[0] * tpw
    mesh = plsc.VectorSubcoreMesh(core_axis_name="c", subcore_axis_name="s")
    f32 = jnp.float32
    i32 = jnp.int32
    run = pl.kernel(
        _sc_topk_body,
        mesh=mesh,
        out_type=[
            jax.ShapeDtypeStruct((n_tok,), f32),
            jax.ShapeDtypeStruct((n_tok,), f32),
            jax.ShapeDtypeStruct((n_tok,), i32),
            jax.ShapeDtypeStruct((n_tok,), i32),
        ],
        scratch_types=[
            pltpu.VMEM((_E, tpw), f32),
            pltpu.VMEM((tpw,), f32),
            pltpu.VMEM((tpw,), f32),
            pltpu.VMEM((tpw,), i32),
            pltpu.VMEM((tpw,), i32),
        ],
    )
    return run(logits_t)


def _fused_body(x_ref, w_ref, b_ref, w_out_ref, i_out_ref):
    x = x_ref[...]
    w = w_ref[...]
    logits = jax.lax.dot_general(
        w, x, (((1,), (1,)), ((), ())), preferred_element_type=jnp.float32
    )
    logits = logits + b_ref[...]
    iota = jax.lax.broadcasted_iota(jnp.int32, logits.shape, 0)
    m1 = jnp.max(logits, axis=0, keepdims=True)
    i1 = jnp.min(jnp.where(logits == m1, iota, _E), axis=0, keepdims=True)
    masked = jnp.where(iota == i1, -jnp.inf, logits)
    m2 = jnp.max(masked, axis=0, keepdims=True)
    i2 = jnp.min(jnp.where(masked == m2, iota, _E), axis=0, keepdims=True)
    ex = jnp.exp(m2 - m1)
    w1 = 1.0 / (1.0 + ex)
    w_out_ref[...] = jnp.concatenate([w1, 1.0 - w1], axis=0)
    i_out_ref[...] = jnp.concatenate([i1, i2], axis=0)


def _tc_fused(x, weight, bias):
    n_tok = x.shape[0]
    b2 = bias.reshape(_E, 1)
    grid = (n_tok // _BM,)
    return pl.pallas_call(
        _fused_body,
        grid=grid,
        in_specs=[
            pl.BlockSpec((_BM, _HIDDEN), lambda i: (i, 0)),
            pl.BlockSpec((_E, _HIDDEN), lambda i: (0, 0)),
            pl.BlockSpec((_E, 1), lambda i: (0, 0)),
        ],
        out_specs=[
            pl.BlockSpec((2, _BM), lambda i: (0, i)),
            pl.BlockSpec((2, _BM), lambda i: (0, i)),
        ],
        out_shape=[
            jax.ShapeDtypeStruct((2, n_tok), jnp.float32),
            jax.ShapeDtypeStruct((2, n_tok), jnp.int32),
        ],
        compiler_params=pltpu.CompilerParams(
            dimension_semantics=("arbitrary",),
        ),
    )(x, weight, b2)


def kernel(x, weight, bias):
    n_tok = x.shape[0]
    half = n_tok // 2
    x0 = lax.slice(x, (0, 0), (half, _HIDDEN))
    x1 = lax.slice(x, (half, 0), (n_tok, _HIDDEN))
    logits_t1 = _tc_logits(x1, weight, bias)
    w1b_, w2b_, i1b_, i2b_ = _sc_topk(logits_t1)
    wi0, ii0 = _tc_fused(x0, weight, bias)
    top_w = jnp.concatenate(
        [wi0.T, jnp.stack([w1b_, w2b_], axis=1)], axis=0)
    top_i = jnp.concatenate(
        [ii0.T, jnp.stack([i1b_, i2b_], axis=1)], axis=0)
    return (top_w, top_i)


# speedup vs baseline: 2.0229x; 2.0229x over previous
import functools

import jax
import jax.numpy as jnp
from jax import lax
from jax.experimental import pallas as pl
from jax.experimental.pallas import tpu as pltpu
from jax.experimental.pallas import tpu_sc as plsc

_HIDDEN = 768
_E = 64
_BM = 4096
_L = 16
_NW = 32


def _logits_body(x_ref, w_ref, b_ref, out_ref):
    w = w_ref[...]
    b = b_ref[...]
    spw = out_ref.shape[0]
    tpw = out_ref.shape[2]
    for j in range(spw):
        xj = x_ref[pl.ds(j * tpw, tpw), :]
        logits = jax.lax.dot_general(
            w, xj, (((1,), (1,)), ((), ())),
            preferred_element_type=jnp.float32,
        )
        out_ref[j] = logits + b


def _tc_logits(x, weight, bias, n_tok, blk_off):
    tpw = n_tok // _NW
    spw = _BM // tpw
    b2 = bias.reshape(_E, 1)
    grid = (n_tok // _BM,)
    return pl.pallas_call(
        _logits_body,
        grid=grid,
        in_specs=[
            pl.BlockSpec((_BM, _HIDDEN), lambda i: (i + blk_off, 0)),
            pl.BlockSpec((_E, _HIDDEN), lambda i: (0, 0)),
            pl.BlockSpec((_E, 1), lambda i: (0, 0)),
        ],
        out_specs=pl.BlockSpec((spw, _E, tpw), lambda i: (i, 0, 0)),
        out_shape=jax.ShapeDtypeStruct((_NW, _E, tpw), jnp.float32),
        compiler_params=pltpu.CompilerParams(
            dimension_semantics=("arbitrary",),
        ),
    )(x, weight, b2)


def _sc_topk_body(lg_hbm, w1_hbm, w2_hbm, i1_hbm, i2_hbm,
                  buf, w1b, w2b, i1b, i2b):
    nc = 2
    wid = lax.axis_index("s") * nc + lax.axis_index("c")
    tpw = buf.shape[1]
    base = wid * tpw
    pltpu.sync_copy(lg_hbm.at[wid, :, :], buf)

    gb = 4

    def group(g, _):
        offs = [g * (gb * _L) + j * _L for j in range(gb)]
        m1 = [buf[0, pl.ds(o, _L)] for o in offs]
        i1 = [jnp.zeros((_L,), jnp.int32) for _ in offs]
        m2 = [jnp.full((_L,), -jnp.inf, jnp.float32) for _ in offs]
        i2 = [jnp.zeros((_L,), jnp.int32) for _ in offs]
        for e in range(1, _E):
            ev = jnp.full((_L,), e, jnp.int32)
            for j in range(gb):
                v = buf[e, pl.ds(offs[j], _L)]
                gt1 = v > m1[j]
                gt2 = v > m2[j]
                m2[j] = jnp.where(gt1, m1[j], jnp.where(gt2, v, m2[j]))
                i2[j] = jnp.where(gt1, i1[j], jnp.where(gt2, ev, i2[j]))
                m1[j] = jnp.where(gt1, v, m1[j])
                i1[j] = jnp.where(gt1, ev, i1[j])
        for j in range(gb):
            ex = jnp.exp(m2[j] - m1[j])
            w1 = 1.0 / (1.0 + ex)
            w1b[pl.ds(offs[j], _L)] = w1
            w2b[pl.ds(offs[j], _L)] = 1.0 - w1
            i1b[pl.ds(offs[j], _L)] = i1[j]
            i2b[pl.ds(offs[j], _L)] = i2[j]
        return 0

    lax.fori_loop(0, tpw // (gb * _L), group, 0)
    pltpu.sync_copy(w1b, w1_hbm.at[pl.ds(base, tpw)])
    pltpu.sync_copy(w2b, w2_hbm.at[pl.ds(base, tpw)])
    pltpu.sync_copy(i1b, i1_hbm.at[pl.ds(base, tpw)])
    pltpu.sync_copy(i2b, i2_hbm.at[pl.ds(base, tpw)])


def _sc_topk(logits_t):
    tpw = logits_t.shape[2]
    n_tok = logits_t.shape[0] * tpw
    mesh = plsc.VectorSubcoreMesh(core_axis_name="c", subcore_axis_name="s")
    f32 = jnp.float32
    i32 = jnp.int32
    run = pl.kernel(
        _sc_topk_body,
        mesh=mesh,
        out_type=[
            jax.ShapeDtypeStruct((n_tok,), f32),
            jax.ShapeDtypeStruct((n_tok,), f32),
            jax.ShapeDtypeStruct((n_tok,), i32),
            jax.ShapeDtypeStruct((n_tok,), i32),
        ],
        scratch_types=[
            pltpu.VMEM((_E, tpw), f32),
            pltpu.VMEM((tpw,), f32),
            pltpu.VMEM((tpw,), f32),
            pltpu.VMEM((tpw,), i32),
            pltpu.VMEM((tpw,), i32),
        ],
    )
    return run(logits_t)


def _fused_body(x_ref, w_ref, b_ref, w_out_ref, i_out_ref):
    x = x_ref[...]
    w = w_ref[...]
    logits = jax.lax.dot_general(
        w, x, (((1,), (1,)), ((), ())), preferred_element_type=jnp.float32
    )
    logits = logits + b_ref[...]
    iota = jax.lax.broadcasted_iota(jnp.int32, logits.shape, 0)
    m1 = jnp.max(logits, axis=0, keepdims=True)
    i1 = jnp.min(jnp.where(logits == m1, iota, _E), axis=0, keepdims=True)
    masked = jnp.where(iota == i1, -jnp.inf, logits)
    m2 = jnp.max(masked, axis=0, keepdims=True)
    i2 = jnp.min(jnp.where(masked == m2, iota, _E), axis=0, keepdims=True)
    ex = jnp.exp(m2 - m1)
    w1 = 1.0 / (1.0 + ex)
    w_out_ref[...] = jnp.concatenate([w1, 1.0 - w1], axis=0)
    i_out_ref[...] = jnp.concatenate([i1, i2], axis=0)


def _tc_fused(x, weight, bias, n_tok, blk_off):
    b2 = bias.reshape(_E, 1)
    grid = (n_tok // _BM,)
    return pl.pallas_call(
        _fused_body,
        grid=grid,
        in_specs=[
            pl.BlockSpec((_BM, _HIDDEN), lambda i: (i + blk_off, 0)),
            pl.BlockSpec((_E, _HIDDEN), lambda i: (0, 0)),
            pl.BlockSpec((_E, 1), lambda i: (0, 0)),
        ],
        out_specs=[
            pl.BlockSpec((2, _BM), lambda i: (0, i)),
            pl.BlockSpec((2, _BM), lambda i: (0, i)),
        ],
        out_shape=[
            jax.ShapeDtypeStruct((2, n_tok), jnp.float32),
            jax.ShapeDtypeStruct((2, n_tok), jnp.int32),
        ],
        compiler_params=pltpu.CompilerParams(
            dimension_semantics=("arbitrary",),
        ),
    )(x, weight, b2)


def kernel(x, weight, bias):
    n_tok = x.shape[0]
    half = n_tok // 2
    logits_t1 = _tc_logits(x, weight, bias, half, half // _BM)
    w1b_, w2b_, i1b_, i2b_ = _sc_topk(logits_t1)
    wi0, ii0 = _tc_fused(x, weight, bias, half, 0)
    top_w = jnp.concatenate(
        [wi0.T, jnp.stack([w1b_, w2b_], axis=1)], axis=0)
    top_i = jnp.concatenate(
        [ii0.T, jnp.stack([i1b_, i2b_], axis=1)], axis=0)
    return (top_w, top_i)


# speedup vs baseline: 2.0836x; 1.0300x over previous
import functools

import jax
import jax.numpy as jnp
from jax import lax
from jax.experimental import pallas as pl
from jax.experimental.pallas import tpu as pltpu
from jax.experimental.pallas import tpu_sc as plsc

_HIDDEN = 768
_E = 64
_BM = 4096
_L = 16
_NW = 32


def _logits_body(x_ref, w_ref, b_ref, out_ref):
    w = w_ref[...]
    b = b_ref[...]
    spw = out_ref.shape[0]
    tpw = out_ref.shape[2]
    for j in range(spw):
        xj = x_ref[pl.ds(j * tpw, tpw), :]
        logits = jax.lax.dot_general(
            w, xj, (((1,), (1,)), ((), ())),
            preferred_element_type=jnp.float32,
        )
        out_ref[j] = logits + b


def _tc_logits(x, weight, bias, n_tok, blk_off):
    tpw = n_tok // _NW
    spw = _BM // tpw
    b2 = bias.reshape(_E, 1)
    grid = (n_tok // _BM,)
    return pl.pallas_call(
        _logits_body,
        grid=grid,
        in_specs=[
            pl.BlockSpec((_BM, _HIDDEN), lambda i: (i + blk_off, 0)),
            pl.BlockSpec((_E, _HIDDEN), lambda i: (0, 0)),
            pl.BlockSpec((_E, 1), lambda i: (0, 0)),
        ],
        out_specs=pl.BlockSpec((spw, _E, tpw), lambda i: (i, 0, 0)),
        out_shape=jax.ShapeDtypeStruct((_NW, _E, tpw), jnp.float32),
        compiler_params=pltpu.CompilerParams(
            dimension_semantics=("arbitrary",),
        ),
    )(x, weight, b2)


def _sc_topk_body(lg_hbm, w1_hbm, w2_hbm, i1_hbm, i2_hbm,
                  buf, w1b, w2b, i1b, i2b):
    nc = 2
    wid = lax.axis_index("s") * nc + lax.axis_index("c")
    tpw = buf.shape[1]
    base = wid * tpw
    pltpu.sync_copy(lg_hbm.at[wid, :, :], buf)

    gb = 4

    def group(g, _):
        offs = [g * (gb * _L) + j * _L for j in range(gb)]
        m1 = [buf[0, pl.ds(o, _L)] for o in offs]
        i1 = [jnp.zeros((_L,), jnp.int32) for _ in offs]
        m2 = [jnp.full((_L,), -jnp.inf, jnp.float32) for _ in offs]
        i2 = [jnp.zeros((_L,), jnp.int32) for _ in offs]
        for e in range(1, _E):
            ev = jnp.full((_L,), e, jnp.int32)
            for j in range(gb):
                v = buf[e, pl.ds(offs[j], _L)]
                gt1 = v > m1[j]
                gt2 = v > m2[j]
                m2[j] = jnp.where(gt1, m1[j], jnp.where(gt2, v, m2[j]))
                i2[j] = jnp.where(gt1, i1[j], jnp.where(gt2, ev, i2[j]))
                m1[j] = jnp.where(gt1, v, m1[j])
                i1[j] = jnp.where(gt1, ev, i1[j])
        for j in range(gb):
            ex = jnp.exp(m2[j] - m1[j])
            w1 = 1.0 / (1.0 + ex)
            w1b[pl.ds(offs[j], _L)] = w1
            w2b[pl.ds(offs[j], _L)] = 1.0 - w1
            i1b[pl.ds(offs[j], _L)] = i1[j]
            i2b[pl.ds(offs[j], _L)] = i2[j]
        return 0

    lax.fori_loop(0, tpw // (gb * _L), group, 0)
    pltpu.sync_copy(w1b, w1_hbm.at[pl.ds(base, tpw)])
    pltpu.sync_copy(w2b, w2_hbm.at[pl.ds(base, tpw)])
    pltpu.sync_copy(i1b, i1_hbm.at[pl.ds(base, tpw)])
    pltpu.sync_copy(i2b, i2_hbm.at[pl.ds(base, tpw)])


def _sc_topk(logits_t):
    tpw = logits_t.shape[2]
    n_tok = logits_t.shape[0] * tpw
    mesh = plsc.VectorSubcoreMesh(core_axis_name="c", subcore_axis_name="s")
    f32 = jnp.float32
    i32 = jnp.int32
    run = pl.kernel(
        _sc_topk_body,
        mesh=mesh,
        out_type=[
            jax.ShapeDtypeStruct((n_tok,), f32),
            jax.ShapeDtypeStruct((n_tok,), f32),
            jax.ShapeDtypeStruct((n_tok,), i32),
            jax.ShapeDtypeStruct((n_tok,), i32),
        ],
        scratch_types=[
            pltpu.VMEM((_E, tpw), f32),
            pltpu.VMEM((tpw,), f32),
            pltpu.VMEM((tpw,), f32),
            pltpu.VMEM((tpw,), i32),
            pltpu.VMEM((tpw,), i32),
        ],
    )
    return run(logits_t)


def _fused_body(x_ref, w_ref, b_ref, w_out_ref, i_out_ref):
    x = x_ref[...]
    w = w_ref[...]
    logits = jax.lax.dot_general(
        w, x, (((1,), (1,)), ((), ())), preferred_element_type=jnp.float32
    )
    logits = logits + b_ref[...]
    iota = jax.lax.broadcasted_iota(jnp.int32, logits.shape, 0)
    m1 = jnp.max(logits, axis=0, keepdims=True)
    i1 = jnp.min(jnp.where(logits == m1, iota, _E), axis=0, keepdims=True)
    masked = jnp.where(iota == i1, -jnp.inf, logits)
    m2 = jnp.max(masked, axis=0, keepdims=True)
    i2 = jnp.min(jnp.where(masked == m2, iota, _E), axis=0, keepdims=True)
    ex = jnp.exp(m2 - m1)
    w1 = 1.0 / (1.0 + ex)
    w_out_ref[...] = jnp.concatenate([w1, 1.0 - w1], axis=0)
    i_out_ref[...] = jnp.concatenate([i1, i2], axis=0)


def _tc_fused(x, weight, bias, n_tok, blk_off):
    b2 = bias.reshape(_E, 1)
    grid = (n_tok // _BM,)
    return pl.pallas_call(
        _fused_body,
        grid=grid,
        in_specs=[
            pl.BlockSpec((_BM, _HIDDEN), lambda i: (i + blk_off, 0)),
            pl.BlockSpec((_E, _HIDDEN), lambda i: (0, 0)),
            pl.BlockSpec((_E, 1), lambda i: (0, 0)),
        ],
        out_specs=[
            pl.BlockSpec((2, _BM), lambda i: (0, i)),
            pl.BlockSpec((2, _BM), lambda i: (0, i)),
        ],
        out_shape=[
            jax.ShapeDtypeStruct((2, n_tok), jnp.float32),
            jax.ShapeDtypeStruct((2, n_tok), jnp.int32),
        ],
        compiler_params=pltpu.CompilerParams(
            dimension_semantics=("arbitrary",),
        ),
    )(x, weight, b2)


def kernel(x, weight, bias):
    n_tok = x.shape[0]
    sc_share = n_tok // 4
    tc_share = n_tok - sc_share
    logits_t1 = _tc_logits(x, weight, bias, sc_share, tc_share // _BM)
    w1b_, w2b_, i1b_, i2b_ = _sc_topk(logits_t1)
    wi0, ii0 = _tc_fused(x, weight, bias, tc_share, 0)
    top_w = jnp.concatenate(
        [wi0.T, jnp.stack([w1b_, w2b_], axis=1)], axis=0)
    top_i = jnp.concatenate(
        [ii0.T, jnp.stack([i1b_, i2b_], axis=1)], axis=0)
    return (top_w, top_i)


# speedup vs baseline: 2.0848x; 1.0006x over previous
import jax
import jax.numpy as jnp
from jax import lax
from jax.experimental import pallas as pl
from jax.experimental.pallas import tpu as pltpu
from jax.experimental.pallas import tpu_sc as plsc

_HIDDEN = 768
_E = 64
_BM = 4096
_L = 16
_NW = 32


def _logits_body(x_ref, w_ref, b_ref, out_ref):
    w = w_ref[...]
    b = b_ref[...]
    spw = out_ref.shape[0]
    tpw = out_ref.shape[2]
    for j in range(spw):
        xj = x_ref[pl.ds(j * tpw, tpw), :]
        logits = jax.lax.dot_general(
            w, xj, (((1,), (1,)), ((), ())),
            preferred_element_type=jnp.float32,
        )
        out_ref[j] = logits + b


def _tc_logits(x, weight, bias, n_tok, blk_off):
    tpw = n_tok // _NW
    spw = _BM // tpw
    b2 = bias.reshape(_E, 1)
    grid = (n_tok // _BM,)
    return pl.pallas_call(
        _logits_body,
        grid=grid,
        in_specs=[
            pl.BlockSpec((_BM, _HIDDEN), lambda i: (i + blk_off, 0)),
            pl.BlockSpec((_E, _HIDDEN), lambda i: (0, 0)),
            pl.BlockSpec((_E, 1), lambda i: (0, 0)),
        ],
        out_specs=pl.BlockSpec((spw, _E, tpw), lambda i: (i, 0, 0)),
        out_shape=jax.ShapeDtypeStruct((_NW, _E, tpw), jnp.float32),
        compiler_params=pltpu.CompilerParams(
            dimension_semantics=("arbitrary",),
        ),
    )(x, weight, b2)


def _sc_topk_body(lg_hbm, w1_hbm, w2_hbm, i1_hbm, i2_hbm,
                  buf, w1b, w2b, i1b, i2b):
    nc = 2
    wid = lax.axis_index("s") * nc + lax.axis_index("c")
    tpw = buf.shape[1]
    base = wid * tpw
    pltpu.sync_copy(lg_hbm.at[wid, :, :], buf)

    gb = 4

    def group(g, _):
        offs = [g * (gb * _L) + j * _L for j in range(gb)]
        m1 = [buf[0, pl.ds(o, _L)] for o in offs]
        i1 = [jnp.zeros((_L,), jnp.int32) for _ in offs]
        m2 = [jnp.full((_L,), -jnp.inf, jnp.float32) for _ in offs]
        i2 = [jnp.zeros((_L,), jnp.int32) for _ in offs]
        for e in range(1, _E):
            ev = jnp.full((_L,), e, jnp.int32)
            for j in range(gb):
                v = buf[e, pl.ds(offs[j], _L)]
                gt1 = v > m1[j]
                gt2 = v > m2[j]
                m2[j] = jnp.where(gt1, m1[j], jnp.where(gt2, v, m2[j]))
                i2[j] = jnp.where(gt1, i1[j], jnp.where(gt2, ev, i2[j]))
                m1[j] = jnp.where(gt1, v, m1[j])
                i1[j] = jnp.where(gt1, ev, i1[j])
        for j in range(gb):
            ex = jnp.exp(m2[j] - m1[j])
            w1 = 1.0 / (1.0 + ex)
            w1b[pl.ds(offs[j], _L)] = w1
            w2b[pl.ds(offs[j], _L)] = 1.0 - w1
            i1b[pl.ds(offs[j], _L)] = i1[j]
            i2b[pl.ds(offs[j], _L)] = i2[j]
        return 0

    lax.fori_loop(0, tpw // (gb * _L), group, 0)
    pltpu.sync_copy(w1b, w1_hbm.at[pl.ds(base, tpw)])
    pltpu.sync_copy(w2b, w2_hbm.at[pl.ds(base, tpw)])
    pltpu.sync_copy(i1b, i1_hbm.at[pl.ds(base, tpw)])
    pltpu.sync_copy(i2b, i2_hbm.at[pl.ds(base, tpw)])


def _sc_topk(logits_t):
    tpw = logits_t.shape[2]
    n_tok = logits_t.shape[0] * tpw
    mesh = plsc.VectorSubcoreMesh(core_axis_name="c", subcore_axis_name="s")
    f32 = jnp.float32
    i32 = jnp.int32
    run = pl.kernel(
        _sc_topk_body,
        mesh=mesh,
        out_type=[
            jax.ShapeDtypeStruct((n_tok,), f32),
            jax.ShapeDtypeStruct((n_tok,), f32),
            jax.ShapeDtypeStruct((n_tok,), i32),
            jax.ShapeDtypeStruct((n_tok,), i32),
        ],
        scratch_types=[
            pltpu.VMEM((_E, tpw), f32),
            pltpu.VMEM((tpw,), f32),
            pltpu.VMEM((tpw,), f32),
            pltpu.VMEM((tpw,), i32),
            pltpu.VMEM((tpw,), i32),
        ],
    )
    return run(logits_t)


def _fused_body(x_ref, w_ref, b_ref, w_out_ref, i_out_ref):
    x = x_ref[...]
    w = w_ref[...]
    logits = jax.lax.dot_general(
        w, x, (((1,), (1,)), ((), ())), preferred_element_type=jnp.float32
    )
    logits = logits + b_ref[...]
    iota = jax.lax.broadcasted_iota(jnp.int32, logits.shape, 0)
    m1 = jnp.max(logits, axis=0, keepdims=True)
    i1 = jnp.min(jnp.where(logits == m1, iota, _E), axis=0, keepdims=True)
    masked = jnp.where(iota == i1, -jnp.inf, logits)
    m2 = jnp.max(masked, axis=0, keepdims=True)
    i2 = jnp.min(jnp.where(masked == m2, iota, _E), axis=0, keepdims=True)
    ex = jnp.exp(m2 - m1)
    w1 = 1.0 / (1.0 + ex)
    w_out_ref[...] = jnp.concatenate([w1, 1.0 - w1], axis=0)
    i_out_ref[...] = jnp.concatenate([i1, i2], axis=0)


def _tc_fused(x, weight, bias, n_tok, blk_off):
    b2 = bias.reshape(_E, 1)
    grid = (n_tok // _BM,)
    return pl.pallas_call(
        _fused_body,
        grid=grid,
        in_specs=[
            pl.BlockSpec((_BM, _HIDDEN), lambda i: (i + blk_off, 0)),
            pl.BlockSpec((_E, _HIDDEN), lambda i: (0, 0)),
            pl.BlockSpec((_E, 1), lambda i: (0, 0)),
        ],
        out_specs=[
            pl.BlockSpec((2, _BM), lambda i: (0, i)),
            pl.BlockSpec((2, _BM), lambda i: (0, i)),
        ],
        out_shape=[
            jax.ShapeDtypeStruct((2, n_tok), jnp.float32),
            jax.ShapeDtypeStruct((2, n_tok), jnp.int32),
        ],
        compiler_params=pltpu.CompilerParams(
            dimension_semantics=("arbitrary",),
        ),
    )(x, weight, b2)


def kernel(x, weight, bias):
    n_tok = x.shape[0]
    sc_share = n_tok // 4
    tc_share = n_tok - sc_share
    logits_t1 = _tc_logits(x, weight, bias, sc_share, tc_share // _BM)
    w1b_, w2b_, i1b_, i2b_ = _sc_topk(logits_t1)
    wi0, ii0 = _tc_fused(x, weight, bias, tc_share, 0)
    top_w = jnp.concatenate(
        [wi0.T, jnp.stack([w1b_, w2b_], axis=1)], axis=0)
    top_i = jnp.concatenate(
        [ii0.T, jnp.stack([i1b_, i2b_], axis=1)], axis=0)
    return (top_w, top_i)
